# baseline (speedup 1.0000x reference)
import functools

import jax
import jax.numpy as jnp
from jax import lax
from jax.experimental import pallas as pl
from jax.experimental.pallas import tpu as pltpu
from jax.experimental.pallas import tpu_sc as plsc

N = 10000
NP = 10240
E = 320000
D = 128
G = 64
NC = 2
NS = 16
NW = NC * NS
EPW = E // NW
CW = 128
NCH = 80
EPW_PAD = NCH * CW
RPS = NP // NS

_f32 = jnp.float32
_i32 = jnp.int32


def _attn_prep(h, asv, adv):
    a_s = h @ asv
    a_d = h @ adv
    c = jnp.max(a_s) + jnp.max(a_d)
    m = jnp.maximum(c, 0.2 * c)
    es = a_s + a_d
    p_self = jnp.exp(jnp.maximum(es, 0.2 * es) - m)
    return a_s, a_d, p_self, m


def _write_ext(ref, val, pad_val):
    ref[0:N, :] = val
    ref[N:NP, :] = jnp.full((NP - N, val.shape[1]), pad_val, val.dtype)


def _prep_body(x_ref, w_ref, asv_ref, adv_ref,
               h_out, as_out, ad_out, ps_out, m_out):
    h = x_ref[...] @ w_ref[...]
    a_s, a_d, p_self, m = _attn_prep(h, asv_ref[...], adv_ref[...])
    _write_ext(h_out, h, 0.0)
    _write_ext(as_out, a_s, -1e30)
    _write_ext(ad_out, a_d, -1e30)
    _write_ext(ps_out, p_self, 0.0)
    m_out[...] = jnp.full((16, 1), m, _f32)


def _normalize(outp_ref, denp_ref, ps_ref, h_ref, b_ref):
    num = outp_ref[0] + outp_ref[1] + ps_ref[...] * h_ref[...]
    den = denp_ref[0] + denp_ref[1] + ps_ref[...]
    return num / (den + 1e-16) + b_ref[...]


def _combine_body(outp_ref, denp_ref, ps_ref, h_ref, b_ref,
                  w2_ref, asv_ref, adv_ref,
                  h2_out, as_out, ad_out, ps_out, m_out):
    h1 = _normalize(outp_ref, denp_ref, ps_ref, h_ref, b_ref)
    h2 = h1[0:N, :] @ w2_ref[...]
    a_s, a_d, p_self, m = _attn_prep(h2, asv_ref[...], adv_ref[...])
    _write_ext(h2_out, h2, 0.0)
    _write_ext(as_out, a_s, -1e30)
    _write_ext(ad_out, a_d, -1e30)
    _write_ext(ps_out, p_self, 0.0)
    m_out[...] = jnp.full((16, 1), m, _f32)


def _combine2_body(outp_ref, denp_ref, ps_ref, h_ref, b_ref, h2n_out):
    h2n_out[...] = _normalize(outp_ref, denp_ref, ps_ref, h_ref, b_ref)


def _head_body(h2n_ref, maxp_ref, batch_ref,
               gf_ref, wg_ref, bg_ref, wo_ref, bo_ref, out_ref):
    h2 = h2n_ref[...]
    batch = batch_ref[...]
    gids = lax.broadcasted_iota(_i32, (G, NP), 0)
    onehot = (jnp.broadcast_to(batch, (G, NP)) == gids).astype(_f32)
    cnt = jnp.sum(onehot, axis=1, keepdims=True)
    sum_p = jax.lax.dot(onehot, h2,
                        preferred_element_type=_f32)
    mean_p = sum_p / jnp.maximum(cnt, 1.0)
    acc = maxp_ref[0]
    for i in range(1, NW):
        acc = jnp.maximum(acc, maxp_ref[i])
    g_ft = gf_ref[...] @ wg_ref[...] + bg_ref[...]
    flat = jnp.concatenate([mean_p, acc, g_ft], axis=1)
    o = flat @ wo_ref[...] + bo_ref[...]
    mx = jnp.max(o, axis=-1, keepdims=True)
    out_ref[...] = o - mx - jnp.log(jnp.sum(jnp.exp(o - mx), axis=-1,
                                            keepdims=True))


def _sc_edge_body(h_hbm, srcg_hbm, dstg_hbm, as_hbm, ad_hbm, m_hbm,
                  z2_hbm, z1_hbm,
                  outp_hbm, denp_hbm,
                  src_v, dst_v, p_c, asg_v, adg_v, m_v, rows_v,
                  out_sh, den_sh, as_sh, ad_sh, gsem, asem):
    c = lax.axis_index("c")
    s = lax.axis_index("s")
    w = c * NS + s

    base = s * RPS
    pltpu.sync_copy(z2_hbm.at[pl.ds(base, RPS)], out_sh.at[pl.ds(base, RPS)])
    pltpu.sync_copy(z1_hbm.at[pl.ds(base, RPS)], den_sh.at[pl.ds(base, RPS)])

    @pl.when(s == 0)
    def _():
        pltpu.sync_copy(as_hbm, as_sh)
        pltpu.sync_copy(ad_hbm, ad_sh)
    pltpu.sync_copy(srcg_hbm.at[w], src_v)
    pltpu.sync_copy(dstg_hbm.at[w], dst_v)
    pltpu.sync_copy(m_hbm, m_v)
    plsc.subcore_barrier()

    m_vec = m_v[...]
    HCH = NCH // 2

    def start_gathers(l, b):
        return (
            pltpu.async_copy(h_hbm.at[src_v.at[l]], rows_v.at[b], gsem),
            pltpu.async_copy(as_sh.at[src_v.at[l]], asg_v.at[b], asem),
            pltpu.async_copy(ad_sh.at[dst_v.at[l]], adg_v.at[b], asem),
        )

    for h in range(2):
        pltpu.sync_copy(srcg_hbm.at[w * 2 + h], src_v)
        pltpu.sync_copy(dstg_hbm.at[w * 2 + h], dst_v)
        for d_ in start_gathers(0, 0):
            d_.wait()

        def pair(t, _):
            for u in (0, 1):
                l = 2 * t + u
                o = u
                b = 1 - u
                ln = jnp.minimum(l + 1, HCH - 1)
                gd = start_gathers(ln, b)
                for k in range(CW // 16):
                    sl = pl.ds(k * 16, 16)
                    e = asg_v[o, sl] + adg_v[o, sl]
                    e = jnp.maximum(e, 0.2 * e)
                    p_c[o, sl] = jnp.exp(e - m_vec)

                def scale2(e2, _, o=o):
                    for ei in (2 * e2, 2 * e2 + 1):
                        pb = plsc.load_gather(
                            p_c.at[o], [jnp.full((16,), ei, _i32)])
                        for d8 in range(D // 16):
                            sl = pl.ds(d8 * 16, 16)
                            rows_v[o, ei, sl] = rows_v[o, ei, sl] * pb
                    return 0
                lax.fori_loop(0, CW // 2, scale2, 0)

                pltpu.sync_copy(rows_v.at[o], out_sh.at[dst_v.at[l]],
                                add=True)
                pltpu.sync_copy(p_c.at[o], den_sh.at[dst_v.at[l]], add=True)
                for d_ in gd:
                    d_.wait()
            return 0
        lax.fori_loop(0, HCH // 2, pair, 0)

    plsc.subcore_barrier()
    pltpu.sync_copy(out_sh.at[pl.ds(base, RPS)],
                    outp_hbm.at[c, pl.ds(base, RPS)])
    pltpu.sync_copy(den_sh.at[pl.ds(base, RPS)],
                    denp_hbm.at[c, pl.ds(base, RPS)])


_sc_edge = pl.kernel(
    _sc_edge_body,
    out_type=(jax.ShapeDtypeStruct((NC, NP, D), _f32),
              jax.ShapeDtypeStruct((NC, NP), _f32)),
    mesh=plsc.VectorSubcoreMesh(core_axis_name="c", subcore_axis_name="s",
                                num_cores=NC, num_subcores=NS),
    compiler_params=pltpu.CompilerParams(needs_layout_passes=False),
    scratch_types=[
        pltpu.VMEM((NCH // 2, CW), _i32),
        pltpu.VMEM((NCH // 2, CW), _i32),
        pltpu.VMEM((2, CW), _f32),
        pltpu.VMEM((2, CW), _f32),
        pltpu.VMEM((2, CW), _f32),
        pltpu.VMEM((16,), _f32),
        pltpu.VMEM((2, CW, D), _f32),
        pltpu.VMEM_SHARED((NP, D), _f32),
        pltpu.VMEM_SHARED((NP,), _f32),
        pltpu.VMEM_SHARED((NP,), _f32),
        pltpu.VMEM_SHARED((NP,), _f32),
    ] + [pltpu.SemaphoreType.DMA] * 2,
)


RPW = NP // NW


def _sc_pool_body(h_hbm, batch_hbm, maxp_hbm, rows_v, batch_v, acc_v, psem):
    c = lax.axis_index("c")
    s = lax.axis_index("s")
    w = c * NS + s

    cp = pltpu.async_copy(h_hbm.at[pl.ds(w * RPW, RPW)], rows_v, psem)
    pltpu.sync_copy(batch_hbm.at[w], batch_v)

    neg = jnp.full((16,), -jnp.inf, _f32)

    def init_row(g, _):
        for d8 in range(D // 16):
            acc_v[g, pl.ds(d8 * 16, 16)] = neg
        return 0
    lax.fori_loop(0, G + 1, init_row, 0)
    cp.wait()

    def pool_row16(t, _):
        bv = batch_v[pl.ds(t * 16, 16)]
        for k in range(16):
            g = bv[k]
            r = t * 16 + k
            for d8 in range(D // 16):
                sl = pl.ds(d8 * 16, 16)
                acc_v[g, sl] = jnp.maximum(acc_v[g, sl], rows_v[r, sl])
        return 0
    lax.fori_loop(0, RPW // 16, pool_row16, 0)

    pltpu.sync_copy(acc_v.at[pl.ds(0, G)], maxp_hbm.at[w])


_sc_pool = pl.kernel(
    _sc_pool_body,
    out_type=jax.ShapeDtypeStruct((NW, G, D), _f32),
    mesh=plsc.VectorSubcoreMesh(core_axis_name="c", subcore_axis_name="s",
                                num_cores=NC, num_subcores=NS),
    compiler_params=pltpu.CompilerParams(needs_layout_passes=False),
    scratch_types=[
        pltpu.VMEM((RPW, D), _f32),
        pltpu.VMEM((RPW,), _i32),
        pltpu.VMEM((G + 1, D), _f32),
        pltpu.SemaphoreType.DMA,
    ],
)


def _tc(body, out_shape):
    return pl.pallas_call(body, out_shape=out_shape)


def kernel(x, edges_idx, batch_idx, g_features, W1, a_src1, a_dst1, b1,
           W2, a_src2, a_dst2, b2, Wg, bg, Wo, bo):
    src = edges_idx[0].reshape(NW, EPW)
    dst = edges_idx[1].reshape(NW, EPW)
    pad_s = jnp.full((NW, EPW_PAD - EPW), N, _i32)
    pad_d = jnp.zeros((NW, EPW_PAD - EPW), _i32)
    srcg = jnp.concatenate([src, pad_s], axis=1).reshape(NW * 2, NCH // 2, CW)
    dstg = jnp.concatenate([dst, pad_d], axis=1).reshape(NW * 2, NCH // 2, CW)
    z2 = jnp.zeros((NP, D), _f32)
    z1 = jnp.zeros((NP,), _f32)

    ext = jax.ShapeDtypeStruct((NP, 1), _f32)
    prep_out = [jax.ShapeDtypeStruct((NP, D), _f32), ext, ext, ext,
                jax.ShapeDtypeStruct((16, 1), _f32)]

    h1e, as1e, ad1e, ps1, m1 = _tc(_prep_body, prep_out)(
        x, W1, a_src1[:, None], a_dst1[:, None])

    outp1, denp1 = _sc_edge(h1e, srcg, dstg, as1e.reshape(NP),
                            ad1e.reshape(NP), m1.reshape(16), z2, z1)

    h2e, as2e, ad2e, ps2, m2 = _tc(_combine_body, prep_out)(
        outp1, denp1[:, :, None], ps1, h1e, b1[None, :],
        W2, a_src2[:, None], a_dst2[:, None])

    outp2, denp2 = _sc_edge(h2e, srcg, dstg, as2e.reshape(NP),
                            ad2e.reshape(NP), m2.reshape(16), z2, z1)

    h2n = _tc(_combine2_body, jax.ShapeDtypeStruct((NP, D), _f32))(
        outp2, denp2[:, :, None], ps2, h2e, b2[None, :])

    batch_pad = jnp.concatenate(
        [batch_idx, jnp.full((NP - N,), G, _i32)])

    maxp = _sc_pool(h2n, batch_pad.reshape(NW, RPW))

    out = _tc(_head_body, jax.ShapeDtypeStruct((G, 2), _f32))(
        h2n, maxp, batch_pad[None, :],
        g_features, Wg, bg[None, :], Wo, bo[None, :])
    return out


# speedup vs baseline: 1.1111x; 1.1111x over previous
import functools

import jax
import jax.numpy as jnp
from jax import lax
from jax.experimental import pallas as pl
from jax.experimental.pallas import tpu as pltpu
from jax.experimental.pallas import tpu_sc as plsc

N = 10000
NP = 10240
E = 320000
D = 128
G = 64
NC = 2
NS = 16
NW = NC * NS
EPW = E // NW
CW = 128
NCH = 80
EPW_PAD = NCH * CW
RPS = NP // NS

_f32 = jnp.float32
_i32 = jnp.int32


def _attn_prep(h, asv, adv):
    a_s = h @ asv
    a_d = h @ adv
    c = jnp.max(a_s) + jnp.max(a_d)
    m = jnp.maximum(c, 0.2 * c)
    es = a_s + a_d
    p_self = jnp.exp(jnp.maximum(es, 0.2 * es) - m)
    return a_s, a_d, p_self, m


def _write_ext(ref, val, pad_val):
    ref[0:N, :] = val
    ref[N:NP, :] = jnp.full((NP - N, val.shape[1]), pad_val, val.dtype)


def _prep_body(x_ref, w_ref, asv_ref, adv_ref,
               h_out, as_out, ad_out, ps_out, m_out):
    h = x_ref[...] @ w_ref[...]
    a_s, a_d, p_self, m = _attn_prep(h, asv_ref[...], adv_ref[...])
    _write_ext(h_out, h, 0.0)
    _write_ext(as_out, a_s, -1e30)
    _write_ext(ad_out, a_d, -1e30)
    _write_ext(ps_out, p_self, 0.0)
    m_out[...] = jnp.full((16, 1), m, _f32)


def _normalize(outp_ref, denp_ref, ps_ref, h_ref, b_ref):
    num = outp_ref[0] + outp_ref[1] + ps_ref[...] * h_ref[...]
    den = denp_ref[0] + denp_ref[1] + ps_ref[...]
    return num / (den + 1e-16) + b_ref[...]


def _combine_body(outp_ref, denp_ref, ps_ref, h_ref, b_ref,
                  w2_ref, asv_ref, adv_ref,
                  h2_out, as_out, ad_out, ps_out, m_out):
    h1 = _normalize(outp_ref, denp_ref, ps_ref, h_ref, b_ref)
    h2 = h1[0:N, :] @ w2_ref[...]
    a_s, a_d, p_self, m = _attn_prep(h2, asv_ref[...], adv_ref[...])
    _write_ext(h2_out, h2, 0.0)
    _write_ext(as_out, a_s, -1e30)
    _write_ext(ad_out, a_d, -1e30)
    _write_ext(ps_out, p_self, 0.0)
    m_out[...] = jnp.full((16, 1), m, _f32)


def _combine2_body(outp_ref, denp_ref, ps_ref, h_ref, b_ref, h2n_out):
    h2n_out[...] = _normalize(outp_ref, denp_ref, ps_ref, h_ref, b_ref)


def _head_body(h2n_ref, maxp_ref, batch_ref,
               gf_ref, wg_ref, bg_ref, wo_ref, bo_ref, out_ref):
    h2 = h2n_ref[...]
    batch = batch_ref[...]
    gids = lax.broadcasted_iota(_i32, (G, NP), 0)
    onehot = (jnp.broadcast_to(batch, (G, NP)) == gids).astype(_f32)
    cnt = jnp.sum(onehot, axis=1, keepdims=True)
    sum_p = jax.lax.dot(onehot, h2,
                        preferred_element_type=_f32)
    mean_p = sum_p / jnp.maximum(cnt, 1.0)
    acc = maxp_ref[0]
    for i in range(1, NW):
        acc = jnp.maximum(acc, maxp_ref[i])
    g_ft = gf_ref[...] @ wg_ref[...] + bg_ref[...]
    flat = jnp.concatenate([mean_p, acc, g_ft], axis=1)
    o = flat @ wo_ref[...] + bo_ref[...]
    mx = jnp.max(o, axis=-1, keepdims=True)
    out_ref[...] = o - mx - jnp.log(jnp.sum(jnp.exp(o - mx), axis=-1,
                                            keepdims=True))


def _sc_edge_body(h_hbm, srcg_hbm, dstg_hbm, as_hbm, ad_hbm, m_hbm,
                  z2_hbm, z1_hbm,
                  outp_hbm, denp_hbm,
                  src_v, dst_v, p_c, asg_v, adg_v, m_v, rows_v,
                  out_sh, den_sh, as_sh, ad_sh, gsem, asem):
    c = lax.axis_index("c")
    s = lax.axis_index("s")
    w = c * NS + s

    base = s * RPS
    pltpu.sync_copy(z2_hbm.at[pl.ds(base, RPS)], out_sh.at[pl.ds(base, RPS)])
    pltpu.sync_copy(z1_hbm.at[pl.ds(base, RPS)], den_sh.at[pl.ds(base, RPS)])

    @pl.when(s == 0)
    def _():
        pltpu.sync_copy(as_hbm, as_sh)
        pltpu.sync_copy(ad_hbm, ad_sh)
    pltpu.sync_copy(srcg_hbm.at[w], src_v)
    pltpu.sync_copy(dstg_hbm.at[w], dst_v)
    pltpu.sync_copy(m_hbm, m_v)
    plsc.subcore_barrier()

    m_vec = m_v[...]

    def row_chunk(j, _):
        rcp = pltpu.async_copy(h_hbm.at[src_v.at[j]], rows_v, gsem)
        acp = pltpu.async_copy(as_sh.at[src_v.at[j]], asg_v, asem)
        bcp = pltpu.async_copy(ad_sh.at[dst_v.at[j]], adg_v, asem)
        acp.wait()
        bcp.wait()
        for k in range(CW // 16):
            sl = pl.ds(k * 16, 16)
            e = asg_v[sl] + adg_v[sl]
            e = jnp.maximum(e, 0.2 * e)
            p_c[sl] = jnp.exp(e - m_vec)
        rcp.wait()

        def scale2(e2, _):
            for ei in (2 * e2, 2 * e2 + 1):
                pb = plsc.load_gather(p_c, [jnp.full((16,), ei, _i32)])
                for d8 in range(D // 16):
                    sl = pl.ds(d8 * 16, 16)
                    rows_v[ei, sl] = rows_v[ei, sl] * pb
            return 0
        lax.fori_loop(0, CW // 2, scale2, 0)

        sc1 = pltpu.async_copy(rows_v, out_sh.at[dst_v.at[j]], gsem,
                               add=True)
        sc2 = pltpu.async_copy(p_c, den_sh.at[dst_v.at[j]], asem, add=True)
        sc1.wait()
        sc2.wait()
        return 0
    lax.fori_loop(0, NCH, row_chunk, 0)

    plsc.subcore_barrier()
    pltpu.sync_copy(out_sh.at[pl.ds(base, RPS)],
                    outp_hbm.at[c, pl.ds(base, RPS)])
    pltpu.sync_copy(den_sh.at[pl.ds(base, RPS)],
                    denp_hbm.at[c, pl.ds(base, RPS)])


_sc_edge = pl.kernel(
    _sc_edge_body,
    out_type=(jax.ShapeDtypeStruct((NC, NP, D), _f32),
              jax.ShapeDtypeStruct((NC, NP), _f32)),
    mesh=plsc.VectorSubcoreMesh(core_axis_name="c", subcore_axis_name="s",
                                num_cores=NC, num_subcores=NS),
    compiler_params=pltpu.CompilerParams(needs_layout_passes=False),
    scratch_types=[
        pltpu.VMEM((NCH, CW), _i32),
        pltpu.VMEM((NCH, CW), _i32),
        pltpu.VMEM((CW,), _f32),
        pltpu.VMEM((CW,), _f32),
        pltpu.VMEM((CW,), _f32),
        pltpu.VMEM((16,), _f32),
        pltpu.VMEM((CW, D), _f32),
        pltpu.VMEM_SHARED((NP, D), _f32),
        pltpu.VMEM_SHARED((NP,), _f32),
        pltpu.VMEM_SHARED((NP,), _f32),
        pltpu.VMEM_SHARED((NP,), _f32),
    ] + [pltpu.SemaphoreType.DMA] * 2,
)


RPW = NP // NW


def _sc_pool_body(h_hbm, batch_hbm, maxp_hbm, rows_v, batch_v, acc_v, psem):
    c = lax.axis_index("c")
    s = lax.axis_index("s")
    w = c * NS + s

    cp = pltpu.async_copy(h_hbm.at[pl.ds(w * RPW, RPW)], rows_v, psem)
    pltpu.sync_copy(batch_hbm.at[w], batch_v)

    neg = jnp.full((16,), -jnp.inf, _f32)

    def init_row(g, _):
        for d8 in range(D // 16):
            acc_v[g, pl.ds(d8 * 16, 16)] = neg
        return 0
    lax.fori_loop(0, G + 1, init_row, 0)
    cp.wait()

    def pool_row16(t, _):
        bv = batch_v[pl.ds(t * 16, 16)]
        for k in range(16):
            g = bv[k]
            r = t * 16 + k
            for d8 in range(D // 16):
                sl = pl.ds(d8 * 16, 16)
                acc_v[g, sl] = jnp.maximum(acc_v[g, sl], rows_v[r, sl])
        return 0
    lax.fori_loop(0, RPW // 16, pool_row16, 0)

    pltpu.sync_copy(acc_v.at[pl.ds(0, G)], maxp_hbm.at[w])


_sc_pool = pl.kernel(
    _sc_pool_body,
    out_type=jax.ShapeDtypeStruct((NW, G, D), _f32),
    mesh=plsc.VectorSubcoreMesh(core_axis_name="c", subcore_axis_name="s",
                                num_cores=NC, num_subcores=NS),
    compiler_params=pltpu.CompilerParams(needs_layout_passes=False),
    scratch_types=[
        pltpu.VMEM((RPW, D), _f32),
        pltpu.VMEM((RPW,), _i32),
        pltpu.VMEM((G + 1, D), _f32),
        pltpu.SemaphoreType.DMA,
    ],
)


def _tc(body, out_shape):
    return pl.pallas_call(body, out_shape=out_shape)


def kernel(x, edges_idx, batch_idx, g_features, W1, a_src1, a_dst1, b1,
           W2, a_src2, a_dst2, b2, Wg, bg, Wo, bo):
    src = edges_idx[0].reshape(NW, EPW)
    dst = edges_idx[1].reshape(NW, EPW)
    pad_s = jnp.full((NW, EPW_PAD - EPW), N, _i32)
    pad_d = jnp.zeros((NW, EPW_PAD - EPW), _i32)
    srcg = jnp.concatenate([src, pad_s], axis=1).reshape(NW, NCH, CW)
    dstg = jnp.concatenate([dst, pad_d], axis=1).reshape(NW, NCH, CW)
    z2 = jnp.zeros((NP, D), _f32)
    z1 = jnp.zeros((NP,), _f32)

    ext = jax.ShapeDtypeStruct((NP, 1), _f32)
    prep_out = [jax.ShapeDtypeStruct((NP, D), _f32), ext, ext, ext,
                jax.ShapeDtypeStruct((16, 1), _f32)]

    h1e, as1e, ad1e, ps1, m1 = _tc(_prep_body, prep_out)(
        x, W1, a_src1[:, None], a_dst1[:, None])

    outp1, denp1 = _sc_edge(h1e, srcg, dstg, as1e.reshape(NP),
                            ad1e.reshape(NP), m1.reshape(16), z2, z1)

    h2e, as2e, ad2e, ps2, m2 = _tc(_combine_body, prep_out)(
        outp1, denp1[:, :, None], ps1, h1e, b1[None, :],
        W2, a_src2[:, None], a_dst2[:, None])

    outp2, denp2 = _sc_edge(h2e, srcg, dstg, as2e.reshape(NP),
                            ad2e.reshape(NP), m2.reshape(16), z2, z1)

    h2n = _tc(_combine2_body, jax.ShapeDtypeStruct((NP, D), _f32))(
        outp2, denp2[:, :, None], ps2, h2e, b2[None, :])

    batch_pad = jnp.concatenate(
        [batch_idx, jnp.full((NP - N,), G, _i32)])

    maxp = _sc_pool(h2n, batch_pad.reshape(NW, RPW))

    out = _tc(_head_body, jax.ShapeDtypeStruct((G, 2), _f32))(
        h2n, maxp, batch_pad[None, :],
        g_features, Wg, bg[None, :], Wo, bo[None, :])
    return out


# speedup vs baseline: 1.1126x; 1.0014x over previous
import jax
import jax.numpy as jnp
from jax import lax
from jax.experimental import pallas as pl
from jax.experimental.pallas import tpu as pltpu
from jax.experimental.pallas import tpu_sc as plsc

N = 10000
NP = 10240
E = 320000
D = 128
G = 64
NC = 2
NS = 16
NW = NC * NS
EPW = E // NW
CW = 128
NCH = 80
EPW_PAD = NCH * CW
RPS = NP // NS

_f32 = jnp.float32
_i32 = jnp.int32


def _attn_prep(h, asv, adv):
    a_s = h @ asv
    a_d = h @ adv
    c = jnp.max(a_s) + jnp.max(a_d)
    m = jnp.maximum(c, 0.2 * c)
    es = a_s + a_d
    p_self = jnp.exp(jnp.maximum(es, 0.2 * es) - m)
    return a_s, a_d, p_self, m


def _write_ext(ref, val, pad_val):
    ref[0:N, :] = val
    ref[N:NP, :] = jnp.full((NP - N, val.shape[1]), pad_val, val.dtype)


def _prep_body(x_ref, w_ref, asv_ref, adv_ref,
               h_out, as_out, ad_out, ps_out, m_out):
    h = x_ref[...] @ w_ref[...]
    a_s, a_d, p_self, m = _attn_prep(h, asv_ref[...], adv_ref[...])
    _write_ext(h_out, h, 0.0)
    _write_ext(as_out, a_s, -1e30)
    _write_ext(ad_out, a_d, -1e30)
    _write_ext(ps_out, p_self, 0.0)
    m_out[...] = jnp.full((16, 1), m, _f32)


def _normalize(outp_ref, denp_ref, ps_ref, h_ref, b_ref):
    num = outp_ref[0] + outp_ref[1] + ps_ref[...] * h_ref[...]
    den = denp_ref[0] + denp_ref[1] + ps_ref[...]
    return num / (den + 1e-16) + b_ref[...]


def _combine_body(outp_ref, denp_ref, ps_ref, h_ref, b_ref,
                  w2_ref, asv_ref, adv_ref,
                  h2_out, as_out, ad_out, ps_out, m_out):
    h1 = _normalize(outp_ref, denp_ref, ps_ref, h_ref, b_ref)
    h2 = h1[0:N, :] @ w2_ref[...]
    a_s, a_d, p_self, m = _attn_prep(h2, asv_ref[...], adv_ref[...])
    _write_ext(h2_out, h2, 0.0)
    _write_ext(as_out, a_s, -1e30)
    _write_ext(ad_out, a_d, -1e30)
    _write_ext(ps_out, p_self, 0.0)
    m_out[...] = jnp.full((16, 1), m, _f32)


def _combine2_body(outp_ref, denp_ref, ps_ref, h_ref, b_ref, h2n_out):
    h2n_out[...] = _normalize(outp_ref, denp_ref, ps_ref, h_ref, b_ref)


def _head_body(h2n_ref, maxp_ref, batch_ref,
               gf_ref, wg_ref, bg_ref, wo_ref, bo_ref, out_ref):
    h2 = h2n_ref[...]
    batch = batch_ref[...]
    gids = lax.broadcasted_iota(_i32, (G, NP), 0)
    onehot = (jnp.broadcast_to(batch, (G, NP)) == gids).astype(_f32)
    cnt = jnp.sum(onehot, axis=1, keepdims=True)
    sum_p = jax.lax.dot(onehot, h2,
                        preferred_element_type=_f32)
    mean_p = sum_p / jnp.maximum(cnt, 1.0)
    acc = maxp_ref[0]
    for i in range(1, NW):
        acc = jnp.maximum(acc, maxp_ref[i])
    g_ft = gf_ref[...] @ wg_ref[...] + bg_ref[...]
    flat = jnp.concatenate([mean_p, acc, g_ft], axis=1)
    o = flat @ wo_ref[...] + bo_ref[...]
    mx = jnp.max(o, axis=-1, keepdims=True)
    out_ref[...] = o - mx - jnp.log(jnp.sum(jnp.exp(o - mx), axis=-1,
                                            keepdims=True))


def _sc_edge_body(h_hbm, srcg_hbm, dstg_hbm, as_hbm, ad_hbm, m_hbm,
                  z2_hbm, z1_hbm,
                  outp_hbm, denp_hbm,
                  src_v, dst_v, p_c, asg_v, adg_v, m_v, rows_v,
                  out_sh, den_sh, as_sh, ad_sh, gsem, asem):
    c = lax.axis_index("c")
    s = lax.axis_index("s")
    w = c * NS + s

    base = s * RPS
    pltpu.sync_copy(z2_hbm.at[pl.ds(base, RPS)], out_sh.at[pl.ds(base, RPS)])
    pltpu.sync_copy(z1_hbm.at[pl.ds(base, RPS)], den_sh.at[pl.ds(base, RPS)])

    @pl.when(s == 0)
    def _():
        pltpu.sync_copy(as_hbm, as_sh)
        pltpu.sync_copy(ad_hbm, ad_sh)
    pltpu.sync_copy(srcg_hbm.at[w], src_v)
    pltpu.sync_copy(dstg_hbm.at[w], dst_v)
    pltpu.sync_copy(m_hbm, m_v)
    plsc.subcore_barrier()

    m_vec = m_v[...]

    def row_chunk(j, _):
        rcp = pltpu.async_copy(h_hbm.at[src_v.at[j]], rows_v, gsem)
        acp = pltpu.async_copy(as_sh.at[src_v.at[j]], asg_v, asem)
        bcp = pltpu.async_copy(ad_sh.at[dst_v.at[j]], adg_v, asem)
        acp.wait()
        bcp.wait()
        for k in range(CW // 16):
            sl = pl.ds(k * 16, 16)
            e = asg_v[sl] + adg_v[sl]
            e = jnp.maximum(e, 0.2 * e)
            p_c[sl] = jnp.exp(e - m_vec)
        rcp.wait()

        def scale2(e2, _):
            for ei in (2 * e2, 2 * e2 + 1):
                pb = plsc.load_gather(p_c, [jnp.full((16,), ei, _i32)])
                for d8 in range(D // 16):
                    sl = pl.ds(d8 * 16, 16)
                    rows_v[ei, sl] = rows_v[ei, sl] * pb
            return 0
        lax.fori_loop(0, CW // 2, scale2, 0)

        sc1 = pltpu.async_copy(rows_v, out_sh.at[dst_v.at[j]], gsem,
                               add=True)
        sc2 = pltpu.async_copy(p_c, den_sh.at[dst_v.at[j]], asem, add=True)
        sc1.wait()
        sc2.wait()
        return 0
    lax.fori_loop(0, NCH, row_chunk, 0)

    plsc.subcore_barrier()
    pltpu.sync_copy(out_sh.at[pl.ds(base, RPS)],
                    outp_hbm.at[c, pl.ds(base, RPS)])
    pltpu.sync_copy(den_sh.at[pl.ds(base, RPS)],
                    denp_hbm.at[c, pl.ds(base, RPS)])


_sc_edge = pl.kernel(
    _sc_edge_body,
    out_type=(jax.ShapeDtypeStruct((NC, NP, D), _f32),
              jax.ShapeDtypeStruct((NC, NP), _f32)),
    mesh=plsc.VectorSubcoreMesh(core_axis_name="c", subcore_axis_name="s",
                                num_cores=NC, num_subcores=NS),
    compiler_params=pltpu.CompilerParams(needs_layout_passes=False),
    scratch_types=[
        pltpu.VMEM((NCH, CW), _i32),
        pltpu.VMEM((NCH, CW), _i32),
        pltpu.VMEM((CW,), _f32),
        pltpu.VMEM((CW,), _f32),
        pltpu.VMEM((CW,), _f32),
        pltpu.VMEM((16,), _f32),
        pltpu.VMEM((CW, D), _f32),
        pltpu.VMEM_SHARED((NP, D), _f32),
        pltpu.VMEM_SHARED((NP,), _f32),
        pltpu.VMEM_SHARED((NP,), _f32),
        pltpu.VMEM_SHARED((NP,), _f32),
    ] + [pltpu.SemaphoreType.DMA] * 2,
)


RPW = NP // NW


def _sc_pool_body(h_hbm, batch_hbm, maxp_hbm, rows_v, batch_v, acc_v, psem):
    c = lax.axis_index("c")
    s = lax.axis_index("s")
    w = c * NS + s

    cp = pltpu.async_copy(h_hbm.at[pl.ds(w * RPW, RPW)], rows_v, psem)
    pltpu.sync_copy(batch_hbm.at[w], batch_v)

    neg = jnp.full((16,), -jnp.inf, _f32)

    def init_row(g, _):
        for d8 in range(D // 16):
            acc_v[g, pl.ds(d8 * 16, 16)] = neg
        return 0
    lax.fori_loop(0, G + 1, init_row, 0)
    cp.wait()

    def pool_row16(t, _):
        bv = batch_v[pl.ds(t * 16, 16)]
        for k in range(16):
            g = bv[k]
            r = t * 16 + k
            for d8 in range(D // 16):
                sl = pl.ds(d8 * 16, 16)
                acc_v[g, sl] = jnp.maximum(acc_v[g, sl], rows_v[r, sl])
        return 0
    lax.fori_loop(0, RPW // 16, pool_row16, 0)

    pltpu.sync_copy(acc_v.at[pl.ds(0, G)], maxp_hbm.at[w])


_sc_pool = pl.kernel(
    _sc_pool_body,
    out_type=jax.ShapeDtypeStruct((NW, G, D), _f32),
    mesh=plsc.VectorSubcoreMesh(core_axis_name="c", subcore_axis_name="s",
                                num_cores=NC, num_subcores=NS),
    compiler_params=pltpu.CompilerParams(needs_layout_passes=False),
    scratch_types=[
        pltpu.VMEM((RPW, D), _f32),
        pltpu.VMEM((RPW,), _i32),
        pltpu.VMEM((G + 1, D), _f32),
        pltpu.SemaphoreType.DMA,
    ],
)


def _tc(body, out_shape):
    return pl.pallas_call(body, out_shape=out_shape)


def kernel(x, edges_idx, batch_idx, g_features, W1, a_src1, a_dst1, b1,
           W2, a_src2, a_dst2, b2, Wg, bg, Wo, bo):
    src = edges_idx[0].reshape(NW, EPW)
    dst = edges_idx[1].reshape(NW, EPW)
    pad_s = jnp.full((NW, EPW_PAD - EPW), N, _i32)
    pad_d = jnp.zeros((NW, EPW_PAD - EPW), _i32)
    srcg = jnp.concatenate([src, pad_s], axis=1).reshape(NW, NCH, CW)
    dstg = jnp.concatenate([dst, pad_d], axis=1).reshape(NW, NCH, CW)
    z2 = jnp.zeros((NP, D), _f32)
    z1 = jnp.zeros((NP,), _f32)

    ext = jax.ShapeDtypeStruct((NP, 1), _f32)
    prep_out = [jax.ShapeDtypeStruct((NP, D), _f32), ext, ext, ext,
                jax.ShapeDtypeStruct((16, 1), _f32)]

    h1e, as1e, ad1e, ps1, m1 = _tc(_prep_body, prep_out)(
        x, W1, a_src1[:, None], a_dst1[:, None])

    outp1, denp1 = _sc_edge(h1e, srcg, dstg, as1e.reshape(NP),
                            ad1e.reshape(NP), m1.reshape(16), z2, z1)

    h2e, as2e, ad2e, ps2, m2 = _tc(_combine_body, prep_out)(
        outp1, denp1[:, :, None], ps1, h1e, b1[None, :],
        W2, a_src2[:, None], a_dst2[:, None])

    outp2, denp2 = _sc_edge(h2e, srcg, dstg, as2e.reshape(NP),
                            ad2e.reshape(NP), m2.reshape(16), z2, z1)

    h2n = _tc(_combine2_body, jax.ShapeDtypeStruct((NP, D), _f32))(
        outp2, denp2[:, :, None], ps2, h2e, b2[None, :])

    batch_pad = jnp.concatenate(
        [batch_idx, jnp.full((NP - N,), G, _i32)])

    maxp = _sc_pool(h2n, batch_pad.reshape(NW, RPW))

    out = _tc(_head_body, jax.ShapeDtypeStruct((G, 2), _f32))(
        h2n, maxp, batch_pad[None, :],
        g_features, Wg, bg[None, :], Wo, bo[None, :])
    return out
